# Initial kernel scaffold; baseline (speedup 1.0000x reference)
import jax
import jax.numpy as jnp
from jax.experimental import pallas as pl


def kernel(x1, edge_index1, adj1_vals, eps1, x2, edge_index2, adj2_vals, eps2, W_h1, W_mean1, W_logstd1, W_h2, W_mean2, W_std2, W_den, b_den):
    raise NotImplementedError("write your pallas kernel here")


# speedup vs baseline: 1.6532x; 1.6532x over previous
import jax
import jax.numpy as jnp
from jax import lax
from jax.experimental import pallas as pl

_N = 10000
_E = 320000
_D = 128
_H1 = 64
_H2 = 32

_BR = 2000


def _mm_body(x_ref, w_ref, o_ref):
    o_ref[...] = jnp.dot(x_ref[...], w_ref[...],
                         preferred_element_type=jnp.float32)


def _matmul(x, w):
    m, k = x.shape
    n = w.shape[1]
    return pl.pallas_call(
        _mm_body,
        grid=(m // _BR,),
        in_specs=[
            pl.BlockSpec((_BR, k), lambda i: (i, 0)),
            pl.BlockSpec((k, n), lambda i: (0, 0)),
        ],
        out_specs=pl.BlockSpec((_BR, n), lambda i: (i, 0)),
        out_shape=jax.ShapeDtypeStruct((m, n), jnp.float32),
    )(x, w)


def _relu_mm_body(p_ref, w_ref, o_ref):
    h = jnp.maximum(p_ref[...], 0.0)
    o_ref[...] = jnp.dot(h, w_ref[...], preferred_element_type=jnp.float32)


def _relu_matmul(p, w):
    n = w.shape[1]
    return pl.pallas_call(
        _relu_mm_body,
        grid=(_N // _BR,),
        in_specs=[
            pl.BlockSpec((_BR, _H1), lambda i: (i, 0)),
            pl.BlockSpec((_H1, n), lambda i: (0, 0)),
        ],
        out_specs=pl.BlockSpec((_BR, n), lambda i: (i, 0)),
        out_shape=jax.ShapeDtypeStruct((_N, n), jnp.float32),
    )(p, w)


def _z_head_body(p_ref, eps_ref, wden_ref, bden_ref, z_ref, den_ref):
    s = p_ref[...]
    zm = s[:, :_H2]
    zls = s[:, _H2:]
    z_ref[...] = zm + eps_ref[...] * jnp.exp(zls)
    den_ref[...] = (jnp.dot(zm, wden_ref[...],
                            preferred_element_type=jnp.float32)
                    + bden_ref[...])


def _z_and_head(p, eps, w_den, b_den):
    return pl.pallas_call(
        _z_head_body,
        grid=(_N // _BR,),
        in_specs=[
            pl.BlockSpec((_BR, _H1), lambda i: (i, 0)),
            pl.BlockSpec((_BR, _H2), lambda i: (i, 0)),
            pl.BlockSpec((_H2, _H2), lambda i: (0, 0)),
            pl.BlockSpec((_H2,), lambda i: (0,)),
        ],
        out_specs=[
            pl.BlockSpec((_BR, _H2), lambda i: (i, 0)),
            pl.BlockSpec((_BR, _H2), lambda i: (i, 0)),
        ],
        out_shape=[
            jax.ShapeDtypeStruct((_N, _H2), jnp.float32),
            jax.ShapeDtypeStruct((_N, _H2), jnp.float32),
        ],
    )(p, eps, w_den, b_den)


def _z_body(p_ref, eps_ref, z_ref):
    s = p_ref[...]
    z_ref[...] = s[:, :_H2] + eps_ref[...] * jnp.exp(s[:, _H2:])


def _z_only(p, eps):
    return pl.pallas_call(
        _z_body,
        grid=(_N // _BR,),
        in_specs=[
            pl.BlockSpec((_BR, _H1), lambda i: (i, 0)),
            pl.BlockSpec((_BR, _H2), lambda i: (i, 0)),
        ],
        out_specs=pl.BlockSpec((_BR, _H2), lambda i: (i, 0)),
        out_shape=jax.ShapeDtypeStruct((_N, _H2), jnp.float32),
    )(p, eps)


_BI = 1024
_BJ = 2048


def _gram_body(zi_ref, zj_ref, o_ref):
    o_ref[...] = lax.dot_general(
        zi_ref[...], zj_ref[...],
        dimension_numbers=(((1,), (1,)), ((), ())),
        preferred_element_type=jnp.float32,
    )


def _gram(z):
    return pl.pallas_call(
        _gram_body,
        grid=(pl.cdiv(_N, _BI), pl.cdiv(_N, _BJ)),
        in_specs=[
            pl.BlockSpec((_BI, _H2), lambda i, j: (i, 0)),
            pl.BlockSpec((_BJ, _H2), lambda i, j: (j, 0)),
        ],
        out_specs=pl.BlockSpec((_BI, _BJ), lambda i, j: (i, j)),
        out_shape=jax.ShapeDtypeStruct((_N, _N), jnp.float32),
    )(z, z)


def _spmm(src, dst, vals, table):
    msgs = table[src] * vals[:, None]
    return jax.ops.segment_sum(msgs, dst, num_segments=_N)


def kernel(x1, edge_index1, adj1_vals, eps1, x2, edge_index2, adj2_vals, eps2,
           W_h1, W_mean1, W_logstd1, W_h2, W_mean2, W_std2, W_den, b_den):
    Wcat1 = jnp.concatenate([W_mean1, W_logstd1], axis=1)
    Wcat2 = jnp.concatenate([W_mean2, W_std2], axis=1)

    xw1 = _matmul(x1, W_h1)
    p1 = _spmm(edge_index1[0], edge_index1[1], adj1_vals, xw1)
    hc1 = _relu_matmul(p1, Wcat1)
    q1 = _spmm(edge_index1[0], edge_index1[1], adj1_vals, hc1)
    z1, output = _z_and_head(q1, eps1, W_den, b_den)
    rec1 = _gram(z1)

    xw2 = _matmul(x2, W_h2)
    p2 = _spmm(edge_index2[0], edge_index2[1], adj2_vals, xw2)
    hc2 = _relu_matmul(p2, Wcat2)
    q2 = _spmm(edge_index2[0], edge_index2[1], adj2_vals, hc2)
    z2 = _z_only(q2, eps2)
    rec2 = _gram(z2)

    return (rec1, rec2, output)
